# baseline (speedup 1.0000x reference)
import functools

import jax
import jax.numpy as jnp
from jax import lax
from jax.experimental import pallas as pl
from jax.experimental.pallas import tpu as pltpu
from jax.experimental.pallas import tpu_sc as plsc


def _make_gather(V: int, D: int, B: int):
    info = plsc.get_sparse_core_info()
    NC, NS = info.num_cores, info.num_subcores
    NW = NC * NS
    assert B % (8 * NW) == 0 and D % info.num_lanes == 0
    b_per_w = B // NW
    mesh = plsc.VectorSubcoreMesh(core_axis_name="c", subcore_axis_name="s")

    @functools.partial(
        pl.kernel,
        mesh=mesh,
        out_type=jax.ShapeDtypeStruct((B, D), jnp.float32),
        scratch_types=[
            pltpu.VMEM((b_per_w,), jnp.int32),
            pltpu.VMEM((b_per_w, D), jnp.float32),
            pltpu.VMEM((b_per_w, D), jnp.float32),
            pltpu.SemaphoreType.DMA,
        ],
    )
    def gather_k(idx_hbm, table_hbm, pos_hbm, out_hbm, idx_v, rows_v, pos_v, sem):
        wid = lax.axis_index("s") * NC + lax.axis_index("c")
        base = wid * b_per_w
        pltpu.sync_copy(idx_hbm.at[pl.ds(base, b_per_w)], idx_v)
        pos_cp = pltpu.async_copy(pos_hbm.at[pl.ds(base, b_per_w)], pos_v, sem)
        pltpu.async_copy(table_hbm.at[idx_v], rows_v, sem).wait()
        pos_cp.wait()

        nl = info.num_lanes

        def add_row(j, _):
            for k in range(D // nl):
                sl = pl.ds(k * nl, nl)
                rows_v[j, sl] = rows_v[j, sl] + pos_v[j, sl]
            return 0

        lax.fori_loop(0, b_per_w, add_row, 0)
        pltpu.sync_copy(rows_v, out_hbm.at[pl.ds(base, b_per_w)])

    return gather_k


def _head_body(x_ref, w_ref, out_ref):
    out_ref[...] = lax.dot_general(
        w_ref[...], x_ref[...],
        dimension_numbers=(((1,), (1,)), ((), ())),
        preferred_element_type=jnp.float32,
    )


def _head(x, W_out, vt: int = 2000):
    S, D = x.shape
    V = W_out.shape[0]
    nv = pl.cdiv(V, vt)
    return pl.pallas_call(
        _head_body,
        grid=(nv,),
        in_specs=[
            pl.BlockSpec((S, D), lambda i: (0, 0)),
            pl.BlockSpec((vt, D), lambda i: (i, 0)),
        ],
        out_specs=pl.BlockSpec((vt, S), lambda i: (i, 0)),
        out_shape=jax.ShapeDtypeStruct((V, S), jnp.float32),
    )(x, W_out)


def kernel(in_idx, tok_emb, pos_emb, W_out):
    b, s = in_idx.shape
    V, D = tok_emb.shape
    idx_flat = in_idx.reshape(b * s)
    x = _make_gather(V, D, b * s)(idx_flat, tok_emb, pos_emb[:s])
    logits_t = _head(x, W_out)
    return logits_t.T[None]


# speedup vs baseline: 1.0023x; 1.0023x over previous
import functools

import jax
import jax.numpy as jnp
from jax import lax
from jax.experimental import pallas as pl
from jax.experimental.pallas import tpu as pltpu
from jax.experimental.pallas import tpu_sc as plsc


def _make_gather(V: int, D: int, B: int):
    info = plsc.get_sparse_core_info()
    NC, NS = info.num_cores, info.num_subcores
    NW = NC * NS
    assert B % (8 * NW) == 0 and D % info.num_lanes == 0
    b_per_w = B // NW
    mesh = plsc.VectorSubcoreMesh(core_axis_name="c", subcore_axis_name="s")

    @functools.partial(
        pl.kernel,
        mesh=mesh,
        out_type=jax.ShapeDtypeStruct((B, D), jnp.float32),
        scratch_types=[
            pltpu.VMEM((b_per_w,), jnp.int32),
            pltpu.VMEM((b_per_w, D), jnp.float32),
            pltpu.VMEM((b_per_w, D), jnp.float32),
            pltpu.SemaphoreType.DMA,
        ],
    )
    def gather_k(idx_hbm, table_hbm, pos_hbm, out_hbm, idx_v, rows_v, pos_v, sem):
        wid = lax.axis_index("s") * NC + lax.axis_index("c")
        base = wid * b_per_w
        pltpu.sync_copy(idx_hbm.at[pl.ds(base, b_per_w)], idx_v)
        pos_cp = pltpu.async_copy(pos_hbm.at[pl.ds(base, b_per_w)], pos_v, sem)
        pltpu.async_copy(table_hbm.at[idx_v], rows_v, sem).wait()
        pos_cp.wait()

        nl = info.num_lanes

        def add_row(j, _):
            for k in range(D // nl):
                sl = pl.ds(k * nl, nl)
                rows_v[j, sl] = rows_v[j, sl] + pos_v[j, sl]
            return 0

        lax.fori_loop(0, b_per_w, add_row, 0)
        pltpu.sync_copy(rows_v, out_hbm.at[pl.ds(base, b_per_w)])

    return gather_k


def _head_body(x_ref, w_ref, out_ref):
    out_ref[...] = lax.dot_general(
        w_ref[...], x_ref[...],
        dimension_numbers=(((1,), (1,)), ((), ())),
        preferred_element_type=jnp.float32,
    )


def _head(x, W_out, vt: int = 2048):
    S, D = x.shape
    V = W_out.shape[0]
    nv = pl.cdiv(V, vt)
    return pl.pallas_call(
        _head_body,
        grid=(nv,),
        in_specs=[
            pl.BlockSpec((S, D), lambda i: (0, 0)),
            pl.BlockSpec((vt, D), lambda i: (i, 0)),
        ],
        out_specs=pl.BlockSpec((vt, S), lambda i: (i, 0)),
        out_shape=jax.ShapeDtypeStruct((V, S), jnp.float32),
        compiler_params=pltpu.CompilerParams(
            dimension_semantics=("parallel",),
        ),
    )(x, W_out)


def kernel(in_idx, tok_emb, pos_emb, W_out):
    b, s = in_idx.shape
    V, D = tok_emb.shape
    idx_flat = in_idx.reshape(b * s)
    x = _make_gather(V, D, b * s)(idx_flat, tok_emb, pos_emb[:s])
    logits_t = _head(x, W_out)
    return logits_t.T[None]


# speedup vs baseline: 1.0023x; 1.0000x over previous
import functools

import jax
import jax.numpy as jnp
from jax import lax
from jax.experimental import pallas as pl
from jax.experimental.pallas import tpu as pltpu
from jax.experimental.pallas import tpu_sc as plsc


def _make_gather(V: int, D: int, B: int):
    info = plsc.get_sparse_core_info()
    NC, NS = info.num_cores, info.num_subcores
    NW = NC * NS
    assert B % (8 * NW) == 0 and D % info.num_lanes == 0
    b_per_w = B // NW
    mesh = plsc.VectorSubcoreMesh(core_axis_name="c", subcore_axis_name="s")

    @functools.partial(
        pl.kernel,
        mesh=mesh,
        out_type=jax.ShapeDtypeStruct((B, D), jnp.float32),
        scratch_types=[
            pltpu.VMEM((b_per_w,), jnp.int32),
            pltpu.VMEM((b_per_w, D), jnp.float32),
            pltpu.VMEM((b_per_w, D), jnp.float32),
            pltpu.SemaphoreType.DMA,
            pltpu.SemaphoreType.DMA,
            pltpu.SemaphoreType.DMA,
            pltpu.SemaphoreType.DMA,
        ],
    )
    def gather_k(idx_hbm, table_hbm, pos_hbm, out_hbm,
                 idx_v, rows_v, pos_v, sem_a, sem_b, sem_p, sem_o):
        wid = lax.axis_index("s") * NC + lax.axis_index("c")
        base = wid * b_per_w
        nl = info.num_lanes
        G = 4
        C = b_per_w // G

        pltpu.sync_copy(idx_hbm.at[pl.ds(base, b_per_w)], idx_v)
        pos_cp = pltpu.async_copy(pos_hbm.at[pl.ds(base, b_per_w)], pos_v, sem_p)

        sems = [sem_a, sem_b]
        cps = [None] * G
        for g in range(2):
            sl = pl.ds(g * C, C)
            cps[g] = pltpu.async_copy(table_hbm.at[idx_v.at[sl]],
                                      rows_v.at[sl], sems[g % 2])
        pos_cp.wait()

        def add_row(j, _):
            for k in range(D // nl):
                sl = pl.ds(k * nl, nl)
                rows_v[j, sl] = rows_v[j, sl] + pos_v[j, sl]
            return 0

        wbs = [None] * G
        for g in range(G):
            cps[g].wait()
            if g + 2 < G:
                sl = pl.ds((g + 2) * C, C)
                cps[g + 2] = pltpu.async_copy(table_hbm.at[idx_v.at[sl]],
                                              rows_v.at[sl], sems[g % 2])
            lax.fori_loop(g * C, (g + 1) * C, add_row, 0)
            sl = pl.ds(g * C, C)
            wbs[g] = pltpu.async_copy(rows_v.at[sl],
                                      out_hbm.at[pl.ds(base + g * C, C)], sem_o)
        for g in range(G):
            wbs[g].wait()

    return gather_k


def _head_body(x_ref, w_ref, out_ref):
    out_ref[...] = lax.dot_general(
        w_ref[...], x_ref[...],
        dimension_numbers=(((1,), (1,)), ((), ())),
        preferred_element_type=jnp.float32,
    )


def _head(x, W_out, vt: int = 2048):
    S, D = x.shape
    V = W_out.shape[0]
    nv = pl.cdiv(V, vt)
    return pl.pallas_call(
        _head_body,
        grid=(nv,),
        in_specs=[
            pl.BlockSpec((S, D), lambda i: (0, 0)),
            pl.BlockSpec((vt, D), lambda i: (i, 0)),
        ],
        out_specs=pl.BlockSpec((vt, S), lambda i: (i, 0)),
        out_shape=jax.ShapeDtypeStruct((V, S), jnp.float32),
        compiler_params=pltpu.CompilerParams(
            dimension_semantics=("parallel",),
        ),
    )(x, W_out)


def kernel(in_idx, tok_emb, pos_emb, W_out):
    b, s = in_idx.shape
    V, D = tok_emb.shape
    idx_flat = in_idx.reshape(b * s)
    x = _make_gather(V, D, b * s)(idx_flat, tok_emb, pos_emb[:s])
    logits_t = _head(x, W_out)
    return logits_t.T[None]
